# baseline (speedup 1.0000x reference)
import functools

import jax
import jax.numpy as jnp
import numpy as np
from jax import lax
from jax.experimental import pallas as pl
from jax.experimental.pallas import tpu as pltpu
from jax.experimental.pallas import tpu_sc as plsc

_NC = 2
_NS = 16
_NW = _NC * _NS

_FWD_BLOCK = 1024
_COPY_CHUNKS = 32
_COPY_NBUF = 32


def _pallas_copy(x):
    b, d = x.shape
    n = _COPY_CHUNKS
    nbuf = min(_COPY_NBUF, n)
    rows = b // n

    def body(x_ref, o_ref, buf, in_sems, out_sems):
        def in_copy(k):
            return pltpu.make_async_copy(
                x_ref.at[pl.ds(k * rows, rows)], buf.at[k % nbuf], in_sems.at[k % nbuf]
            )

        def out_copy(k):
            return pltpu.make_async_copy(
                buf.at[k % nbuf], o_ref.at[pl.ds(k * rows, rows)], out_sems.at[k % nbuf]
            )

        started_in = 0
        waited_out = 0
        for j in range(nbuf):
            in_copy(j).start()
            started_in += 1
        for i in range(n):
            if started_in < n and waited_out < i:
                out_copy(waited_out).wait()
                waited_out += 1
                in_copy(started_in).start()
                started_in += 1
            in_copy(i).wait()
            out_copy(i).start()
        while waited_out < n:
            out_copy(waited_out).wait()
            waited_out += 1

    return pl.pallas_call(
        body,
        out_shape=jax.ShapeDtypeStruct(x.shape, x.dtype),
        in_specs=[pl.BlockSpec(memory_space=pl.ANY)],
        out_specs=pl.BlockSpec(memory_space=pl.ANY),
        scratch_shapes=[
            pltpu.VMEM((nbuf, rows, d), x.dtype),
            pltpu.SemaphoreType.DMA((nbuf,)),
            pltpu.SemaphoreType.DMA((nbuf,)),
        ],
    )(x)


def _sc_gather(table, idxs):
    b = idxs.shape[0]
    b_per_w = b // _NW
    mesh = plsc.VectorSubcoreMesh(core_axis_name="c", subcore_axis_name="s")

    @functools.partial(
        pl.kernel,
        mesh=mesh,
        out_type=jax.ShapeDtypeStruct((b,), jnp.float32),
        scratch_types=[
            pltpu.VMEM((b_per_w,), jnp.int32),
            pltpu.VMEM((b_per_w,), jnp.float32),
            pltpu.SemaphoreType.DMA,
        ],
    )
    def k(table_hbm, idx_hbm, out_hbm, idx_v, val_v, sem):
        wid = lax.axis_index("s") * _NC + lax.axis_index("c")
        base = wid * b_per_w
        pltpu.sync_copy(idx_hbm.at[pl.ds(base, b_per_w)], idx_v)
        pltpu.async_copy(table_hbm.at[idx_v], val_v, sem).wait()
        pltpu.sync_copy(val_v, out_hbm.at[pl.ds(base, b_per_w)])

    return k(table, idxs)


def _scale_body(g_ref, s_ref, o_ref):
    o_ref[...] = g_ref[...] * s_ref[...]


def _tc_scale(g, scaling):
    b, d = g.shape
    blk = min(_FWD_BLOCK, b)
    return pl.pallas_call(
        _scale_body,
        out_shape=jax.ShapeDtypeStruct((b, d), g.dtype),
        grid=(b // blk,),
        in_specs=[
            pl.BlockSpec((blk, d), lambda i: (i, 0)),
            pl.BlockSpec((blk, 1), lambda i: (i, 0)),
        ],
        out_specs=pl.BlockSpec((blk, d), lambda i: (i, 0)),
    )(g, scaling.reshape(b, 1))


@jax.custom_vjp
def _gsl(x, idxs, grad_scaling_values):
    return _pallas_copy(x)


def _gsl_fwd(x, idxs, grad_scaling_values):
    return _pallas_copy(x), (idxs, grad_scaling_values)


def _gsl_bwd(res, g):
    idxs, grad_scaling_values = res
    scaling = _sc_gather(grad_scaling_values, idxs)
    gx = _tc_scale(g, scaling)
    return (
        gx,
        np.zeros(idxs.shape, dtype=jax.dtypes.float0),
        jnp.zeros_like(grad_scaling_values),
    )


_gsl.defvjp(_gsl_fwd, _gsl_bwd)


def kernel(input, idxs, grad_scaling_values):
    return _gsl(input, idxs, grad_scaling_values)


# speedup vs baseline: 1.1186x; 1.1186x over previous
import functools

import jax
import jax.numpy as jnp
import numpy as np
from jax import lax
from jax.experimental import pallas as pl
from jax.experimental.pallas import tpu as pltpu
from jax.experimental.pallas import tpu_sc as plsc

_NC = 2
_NS = 16
_NW = _NC * _NS

_FWD_BLOCK = 1024
_COPY_CHUNKS = 8
_COPY_NBUF = 8


def _pallas_copy(x):
    b, d = x.shape
    n = _COPY_CHUNKS
    nbuf = min(_COPY_NBUF, n)
    rows = b // n

    def body(x_ref, o_ref, buf, in_sems, out_sems):
        def in_copy(k):
            return pltpu.make_async_copy(
                x_ref.at[pl.ds(k * rows, rows)], buf.at[k % nbuf], in_sems.at[k % nbuf]
            )

        def out_copy(k):
            return pltpu.make_async_copy(
                buf.at[k % nbuf], o_ref.at[pl.ds(k * rows, rows)], out_sems.at[k % nbuf]
            )

        started_in = 0
        waited_out = 0
        for j in range(nbuf):
            in_copy(j).start()
            started_in += 1
        for i in range(n):
            if started_in < n and waited_out < i:
                out_copy(waited_out).wait()
                waited_out += 1
                in_copy(started_in).start()
                started_in += 1
            in_copy(i).wait()
            out_copy(i).start()
        while waited_out < n:
            out_copy(waited_out).wait()
            waited_out += 1

    return pl.pallas_call(
        body,
        out_shape=jax.ShapeDtypeStruct(x.shape, x.dtype),
        in_specs=[pl.BlockSpec(memory_space=pl.ANY)],
        out_specs=pl.BlockSpec(memory_space=pl.ANY),
        scratch_shapes=[
            pltpu.VMEM((nbuf, rows, d), x.dtype),
            pltpu.SemaphoreType.DMA((nbuf,)),
            pltpu.SemaphoreType.DMA((nbuf,)),
        ],
    )(x)


def _sc_gather(table, idxs):
    b = idxs.shape[0]
    b_per_w = b // _NW
    mesh = plsc.VectorSubcoreMesh(core_axis_name="c", subcore_axis_name="s")

    @functools.partial(
        pl.kernel,
        mesh=mesh,
        out_type=jax.ShapeDtypeStruct((b,), jnp.float32),
        scratch_types=[
            pltpu.VMEM((b_per_w,), jnp.int32),
            pltpu.VMEM((b_per_w,), jnp.float32),
            pltpu.SemaphoreType.DMA,
        ],
    )
    def k(table_hbm, idx_hbm, out_hbm, idx_v, val_v, sem):
        wid = lax.axis_index("s") * _NC + lax.axis_index("c")
        base = wid * b_per_w
        pltpu.sync_copy(idx_hbm.at[pl.ds(base, b_per_w)], idx_v)
        pltpu.async_copy(table_hbm.at[idx_v], val_v, sem).wait()
        pltpu.sync_copy(val_v, out_hbm.at[pl.ds(base, b_per_w)])

    return k(table, idxs)


def _scale_body(g_ref, s_ref, o_ref):
    o_ref[...] = g_ref[...] * s_ref[...]


def _tc_scale(g, scaling):
    b, d = g.shape
    blk = min(_FWD_BLOCK, b)
    return pl.pallas_call(
        _scale_body,
        out_shape=jax.ShapeDtypeStruct((b, d), g.dtype),
        grid=(b // blk,),
        in_specs=[
            pl.BlockSpec((blk, d), lambda i: (i, 0)),
            pl.BlockSpec((blk, 1), lambda i: (i, 0)),
        ],
        out_specs=pl.BlockSpec((blk, d), lambda i: (i, 0)),
    )(g, scaling.reshape(b, 1))


@jax.custom_vjp
def _gsl(x, idxs, grad_scaling_values):
    return _pallas_copy(x)


def _gsl_fwd(x, idxs, grad_scaling_values):
    return _pallas_copy(x), (idxs, grad_scaling_values)


def _gsl_bwd(res, g):
    idxs, grad_scaling_values = res
    scaling = _sc_gather(grad_scaling_values, idxs)
    gx = _tc_scale(g, scaling)
    return (
        gx,
        np.zeros(idxs.shape, dtype=jax.dtypes.float0),
        jnp.zeros_like(grad_scaling_values),
    )


_gsl.defvjp(_gsl_fwd, _gsl_bwd)


def kernel(input, idxs, grad_scaling_values):
    return _gsl(input, idxs, grad_scaling_values)


# speedup vs baseline: 1.1190x; 1.0004x over previous
import functools

import jax
import jax.numpy as jnp
import numpy as np
from jax import lax
from jax.experimental import pallas as pl
from jax.experimental.pallas import tpu as pltpu
from jax.experimental.pallas import tpu_sc as plsc

_NC = 2
_NS = 16
_NW = _NC * _NS

_FWD_BLOCK = 1024
_COPY_CHUNKS = 4
_COPY_NBUF = 4


def _pallas_copy(x):
    b, d = x.shape
    n = _COPY_CHUNKS
    nbuf = min(_COPY_NBUF, n)
    rows = b // n

    def body(x_ref, o_ref, buf, in_sems, out_sems):
        def in_copy(k):
            return pltpu.make_async_copy(
                x_ref.at[pl.ds(k * rows, rows)], buf.at[k % nbuf], in_sems.at[k % nbuf]
            )

        def out_copy(k):
            return pltpu.make_async_copy(
                buf.at[k % nbuf], o_ref.at[pl.ds(k * rows, rows)], out_sems.at[k % nbuf]
            )

        started_in = 0
        waited_out = 0
        for j in range(nbuf):
            in_copy(j).start()
            started_in += 1
        for i in range(n):
            if started_in < n and waited_out < i:
                out_copy(waited_out).wait()
                waited_out += 1
                in_copy(started_in).start()
                started_in += 1
            in_copy(i).wait()
            out_copy(i).start()
        while waited_out < n:
            out_copy(waited_out).wait()
            waited_out += 1

    return pl.pallas_call(
        body,
        out_shape=jax.ShapeDtypeStruct(x.shape, x.dtype),
        in_specs=[pl.BlockSpec(memory_space=pl.ANY)],
        out_specs=pl.BlockSpec(memory_space=pl.ANY),
        scratch_shapes=[
            pltpu.VMEM((nbuf, rows, d), x.dtype),
            pltpu.SemaphoreType.DMA((nbuf,)),
            pltpu.SemaphoreType.DMA((nbuf,)),
        ],
    )(x)


def _sc_gather(table, idxs):
    b = idxs.shape[0]
    b_per_w = b // _NW
    mesh = plsc.VectorSubcoreMesh(core_axis_name="c", subcore_axis_name="s")

    @functools.partial(
        pl.kernel,
        mesh=mesh,
        out_type=jax.ShapeDtypeStruct((b,), jnp.float32),
        scratch_types=[
            pltpu.VMEM((b_per_w,), jnp.int32),
            pltpu.VMEM((b_per_w,), jnp.float32),
            pltpu.SemaphoreType.DMA,
        ],
    )
    def k(table_hbm, idx_hbm, out_hbm, idx_v, val_v, sem):
        wid = lax.axis_index("s") * _NC + lax.axis_index("c")
        base = wid * b_per_w
        pltpu.sync_copy(idx_hbm.at[pl.ds(base, b_per_w)], idx_v)
        pltpu.async_copy(table_hbm.at[idx_v], val_v, sem).wait()
        pltpu.sync_copy(val_v, out_hbm.at[pl.ds(base, b_per_w)])

    return k(table, idxs)


def _scale_body(g_ref, s_ref, o_ref):
    o_ref[...] = g_ref[...] * s_ref[...]


def _tc_scale(g, scaling):
    b, d = g.shape
    blk = min(_FWD_BLOCK, b)
    return pl.pallas_call(
        _scale_body,
        out_shape=jax.ShapeDtypeStruct((b, d), g.dtype),
        grid=(b // blk,),
        in_specs=[
            pl.BlockSpec((blk, d), lambda i: (i, 0)),
            pl.BlockSpec((blk, 1), lambda i: (i, 0)),
        ],
        out_specs=pl.BlockSpec((blk, d), lambda i: (i, 0)),
    )(g, scaling.reshape(b, 1))


@jax.custom_vjp
def _gsl(x, idxs, grad_scaling_values):
    return _pallas_copy(x)


def _gsl_fwd(x, idxs, grad_scaling_values):
    return _pallas_copy(x), (idxs, grad_scaling_values)


def _gsl_bwd(res, g):
    idxs, grad_scaling_values = res
    scaling = _sc_gather(grad_scaling_values, idxs)
    gx = _tc_scale(g, scaling)
    return (
        gx,
        np.zeros(idxs.shape, dtype=jax.dtypes.float0),
        jnp.zeros_like(grad_scaling_values),
    )


_gsl.defvjp(_gsl_fwd, _gsl_bwd)


def kernel(input, idxs, grad_scaling_values):
    return _gsl(input, idxs, grad_scaling_values)


# speedup vs baseline: 1.1729x; 1.0482x over previous
import functools

import jax
import jax.numpy as jnp
import numpy as np
from jax import lax
from jax.experimental import pallas as pl
from jax.experimental.pallas import tpu as pltpu
from jax.experimental.pallas import tpu_sc as plsc

_NC = 2
_NS = 16
_NW = _NC * _NS

_FWD_BLOCK = 1024
_COPY_CHUNK_ROWS = (1024, 2048, 5120, 5120, 2048, 1024)


def _pallas_copy(x):
    b, d = x.shape
    sizes = _COPY_CHUNK_ROWS
    assert sum(sizes) == b
    offs = [sum(sizes[:i]) for i in range(len(sizes))]
    n = len(sizes)

    def body(x_ref, o_ref, buf, in_sems, out_sems):
        def in_copy(k):
            return pltpu.make_async_copy(
                x_ref.at[pl.ds(offs[k], sizes[k])],
                buf.at[pl.ds(offs[k], sizes[k])],
                in_sems.at[k],
            )

        def out_copy(k):
            return pltpu.make_async_copy(
                buf.at[pl.ds(offs[k], sizes[k])],
                o_ref.at[pl.ds(offs[k], sizes[k])],
                out_sems.at[k],
            )

        for k in range(n):
            in_copy(k).start()
        for k in range(n):
            in_copy(k).wait()
            out_copy(k).start()
        for k in range(n):
            out_copy(k).wait()

    return pl.pallas_call(
        body,
        out_shape=jax.ShapeDtypeStruct(x.shape, x.dtype),
        in_specs=[pl.BlockSpec(memory_space=pl.ANY)],
        out_specs=pl.BlockSpec(memory_space=pl.ANY),
        scratch_shapes=[
            pltpu.VMEM((b, d), x.dtype),
            pltpu.SemaphoreType.DMA((n,)),
            pltpu.SemaphoreType.DMA((n,)),
        ],
    )(x)


def _sc_gather(table, idxs):
    b = idxs.shape[0]
    b_per_w = b // _NW
    mesh = plsc.VectorSubcoreMesh(core_axis_name="c", subcore_axis_name="s")

    @functools.partial(
        pl.kernel,
        mesh=mesh,
        out_type=jax.ShapeDtypeStruct((b,), jnp.float32),
        scratch_types=[
            pltpu.VMEM((b_per_w,), jnp.int32),
            pltpu.VMEM((b_per_w,), jnp.float32),
            pltpu.SemaphoreType.DMA,
        ],
    )
    def k(table_hbm, idx_hbm, out_hbm, idx_v, val_v, sem):
        wid = lax.axis_index("s") * _NC + lax.axis_index("c")
        base = wid * b_per_w
        pltpu.sync_copy(idx_hbm.at[pl.ds(base, b_per_w)], idx_v)
        pltpu.async_copy(table_hbm.at[idx_v], val_v, sem).wait()
        pltpu.sync_copy(val_v, out_hbm.at[pl.ds(base, b_per_w)])

    return k(table, idxs)


def _scale_body(g_ref, s_ref, o_ref):
    o_ref[...] = g_ref[...] * s_ref[...]


def _tc_scale(g, scaling):
    b, d = g.shape
    blk = min(_FWD_BLOCK, b)
    return pl.pallas_call(
        _scale_body,
        out_shape=jax.ShapeDtypeStruct((b, d), g.dtype),
        grid=(b // blk,),
        in_specs=[
            pl.BlockSpec((blk, d), lambda i: (i, 0)),
            pl.BlockSpec((blk, 1), lambda i: (i, 0)),
        ],
        out_specs=pl.BlockSpec((blk, d), lambda i: (i, 0)),
    )(g, scaling.reshape(b, 1))


@jax.custom_vjp
def _gsl(x, idxs, grad_scaling_values):
    return _pallas_copy(x)


def _gsl_fwd(x, idxs, grad_scaling_values):
    return _pallas_copy(x), (idxs, grad_scaling_values)


def _gsl_bwd(res, g):
    idxs, grad_scaling_values = res
    scaling = _sc_gather(grad_scaling_values, idxs)
    gx = _tc_scale(g, scaling)
    return (
        gx,
        np.zeros(idxs.shape, dtype=jax.dtypes.float0),
        jnp.zeros_like(grad_scaling_values),
    )


_gsl.defvjp(_gsl_fwd, _gsl_bwd)


def kernel(input, idxs, grad_scaling_values):
    return _gsl(input, idxs, grad_scaling_values)
